# baseline (speedup 1.0000x reference)
import functools

import jax
import jax.numpy as jnp
from jax import lax
from jax.experimental import pallas as pl
from jax.experimental.pallas import tpu as pltpu
from jax.experimental.pallas import tpu_sc as plsc

_POS_RFACTOR = 8
_POS_SHIFT = 3
_NC = 2
_NS = 16
_L = 16


def _sc_gather(pos_h, table_h, table_w, hx):
    rows = 16
    n_active = hx // rows
    d = table_h.shape[1]
    hp_w = pos_h.shape[1]
    pos_flat = pos_h.reshape(-1)

    @functools.partial(
        pl.kernel,
        out_type=(
            jax.ShapeDtypeStruct((hx, d), jnp.float32),
            jax.ShapeDtypeStruct((hx, d), jnp.float32),
        ),
        mesh=plsc.VectorSubcoreMesh(core_axis_name="c", subcore_axis_name="s"),
        scratch_types=[
            pltpu.VMEM((rows,), jnp.int32),
            pltpu.VMEM((rows,), jnp.int32),
            pltpu.VMEM((rows, d), jnp.float32),
            pltpu.SemaphoreType.DMA,
        ],
    )
    def body(pos_hbm, th_hbm, tw_hbm, eh_hbm, ew_hbm, vals_v, idx_v, rows_v, sem):
        wid = lax.axis_index("s") * _NC + lax.axis_index("c")

        @pl.when(wid < n_active)
        def _():
            base = wid * rows
            offs = (2 * base + 2 * lax.iota(jnp.int32, _L)) * hp_w
            pltpu.async_copy(pos_hbm.at[offs], vals_v, sem).wait()
            idx_v[...] = lax.shift_right_logical(vals_v[...], _POS_SHIFT)
            pltpu.async_copy(th_hbm.at[idx_v], rows_v, sem).wait()
            pltpu.sync_copy(rows_v, eh_hbm.at[pl.ds(base, rows)])
            pltpu.async_copy(tw_hbm.at[idx_v], rows_v, sem).wait()
            pltpu.sync_copy(rows_v, ew_hbm.at[pl.ds(base, rows)])

    return body(pos_flat, table_h, table_w)


def _tc_combine(x, eh, ew, cb):
    b, c, h, w = x.shape
    nb = c // cb
    eh_t = eh.T.reshape(nb, cb, h)
    ew_t = ew.T.reshape(nb, cb, w)

    def body(x_ref, eh_ref, ew_ref, o_ref):
        et_h = eh_ref[0]
        et_w = ew_ref[0]
        pos = et_h[:, :, None] * et_w[:, None, :]
        o_ref[...] = x_ref[...] + pos[None]

    return pl.pallas_call(
        body,
        grid=(b, nb),
        in_specs=[
            pl.BlockSpec((1, cb, h, w), lambda bi, ci: (bi, ci, 0, 0)),
            pl.BlockSpec((1, cb, h), lambda bi, ci: (ci, 0, 0)),
            pl.BlockSpec((1, cb, w), lambda bi, ci: (ci, 0, 0)),
        ],
        out_specs=pl.BlockSpec((1, cb, h, w), lambda bi, ci: (bi, ci, 0, 0)),
        out_shape=jax.ShapeDtypeStruct(x.shape, x.dtype),
    )(x, eh_t, ew_t)


def kernel(x, pos_h, pos_w, table_h, table_w):
    del pos_w
    eh, ew = _sc_gather(pos_h.astype(jnp.int32), table_h, table_w, x.shape[2])
    return _tc_combine(x, eh, ew, cb=32)
